# Initial kernel scaffold; baseline (speedup 1.0000x reference)
import jax
import jax.numpy as jnp
from jax.experimental import pallas as pl


def kernel(x, edge_index, W, b, prelu_w):
    raise NotImplementedError("write your pallas kernel here")


# speedup vs baseline: 10.5280x; 10.5280x over previous
import functools

import jax
import jax.numpy as jnp
from jax import lax
from jax.experimental import pallas as pl
from jax.experimental.pallas import tpu as pltpu
from jax.experimental.pallas import tpu_sc as plsc

NC = 2
NS = 16
NW = NC * NS
LANES = 16


def _cdiv(a, b):
    return (a + b - 1) // b


def _deg_call(colf, zero_h, idrows):
    EC = colf.shape[1]
    mesh = plsc.VectorSubcoreMesh(core_axis_name="c", subcore_axis_name="s")

    @functools.partial(
        pl.kernel,
        out_type=jax.ShapeDtypeStruct((NC, 128, 128), jnp.float32),
        mesh=mesh,
        scratch_types=[
            pltpu.VMEM((EC,), jnp.int32),
            pltpu.VMEM((128, 128), jnp.float32),
            pltpu.VMEM((1, 128), jnp.int32),
            pltpu.VMEM_SHARED((128, 128), jnp.float32),
        ],
        compiler_params=pltpu.CompilerParams(needs_layout_passes=False),
    )
    def deg_kernel(colf_hbm, zero_hbm, idr_hbm, deg_hbm,
                   col_v, hist_v, idr_v, acc_sh):
        cid = lax.axis_index("c")
        sid = lax.axis_index("s")
        w = cid * NS + sid
        pltpu.sync_copy(colf_hbm.at[w], col_v)
        pltpu.sync_copy(zero_hbm, hist_v)
        pltpu.sync_copy(idr_hbm, idr_v)
        pltpu.sync_copy(zero_hbm.at[pl.ds(0, 8)], acc_sh.at[pl.ds(sid * 8, 8)])

        ones16 = jnp.full((16,), 1.0, jnp.float32)

        def body(g, carry):
            idx16 = col_v[pl.ds(g * 16, 16)]
            hi = lax.shift_right_logical(idx16, 7)
            lo = lax.bitwise_and(idx16, 127)
            plsc.addupdate_scatter(hist_v, [hi, lo], ones16)
            return carry

        lax.fori_loop(0, EC // 16, body, 0)
        plsc.subcore_barrier()
        pltpu.sync_copy(hist_v, acc_sh.at[idr_v.at[0]], add=True)
        plsc.subcore_barrier()
        pltpu.sync_copy(acc_sh.at[pl.ds(sid * 8, 8)],
                        deg_hbm.at[cid, pl.ds(sid * 8, 8)])

    return deg_kernel(colf, zero_h, idrows)


def _msg_call(hs, rowr, colr, zrow, NP, D):
    EB = rowr.shape[1]
    SLAB = NP // NS
    mesh = plsc.VectorSubcoreMesh(core_axis_name="c", subcore_axis_name="s")

    @functools.partial(
        pl.kernel,
        out_type=jax.ShapeDtypeStruct((NC, NP, D), jnp.float32),
        mesh=mesh,
        scratch_types=[
            pltpu.VMEM((EB, 128), jnp.int32),
            pltpu.VMEM((EB, 128), jnp.int32),
            pltpu.VMEM((128, D), jnp.float32),
            pltpu.VMEM((128, D), jnp.float32),
            pltpu.VMEM_SHARED((NP, D), jnp.float32),
            pltpu.SemaphoreType.DMA,
            pltpu.SemaphoreType.DMA,
        ],
    )
    def msg_kernel(hs_hbm, rowr_hbm, colr_hbm, zrow_hbm, out_hbm,
                   row_v, col_v, msg_a, msg_b, acc_sh, sem_a, sem_b):
        cid = lax.axis_index("c")
        sid = lax.axis_index("s")
        w = cid * NS + sid
        pltpu.sync_copy(rowr_hbm.at[w], row_v)
        pltpu.sync_copy(colr_hbm.at[w], col_v)
        pltpu.sync_copy(zrow_hbm, acc_sh.at[pl.ds(sid * SLAB, SLAB)])
        plsc.subcore_barrier()

        def body(j, carry):
            pltpu.async_copy(hs_hbm.at[row_v.at[j]], msg_a, sem_a).wait()
            pltpu.sync_copy(msg_a, acc_sh.at[col_v.at[j]], add=True)
            return carry

        lax.fori_loop(0, EB, body, 0)
        plsc.subcore_barrier()
        pltpu.sync_copy(acc_sh.at[pl.ds(sid * SLAB, SLAB)],
                        out_hbm.at[cid, pl.ds(sid * SLAB, SLAB)])

    return msg_kernel(hs, rowr, colr, zrow)


def _prescale_call(xp, W, d0, d1, BM):
    NP, D = xp.shape

    def body(x_ref, w_ref, d0_ref, d1_ref, hs_ref, dinv_ref):
        deg = d0_ref[...] + d1_ref[...] + 1.0
        dinv = lax.rsqrt(deg)
        h = jnp.dot(x_ref[...], w_ref[...], preferred_element_type=jnp.float32,
                    precision=lax.Precision.HIGHEST)
        hs_ref[...] = h * dinv
        dinv_ref[...] = dinv

    return pl.pallas_call(
        body,
        grid=(NP // BM,),
        in_specs=[
            pl.BlockSpec((BM, D), lambda i: (i, 0)),
            pl.BlockSpec((D, D), lambda i: (0, 0)),
            pl.BlockSpec((BM, 1), lambda i: (i, 0)),
            pl.BlockSpec((BM, 1), lambda i: (i, 0)),
        ],
        out_specs=[
            pl.BlockSpec((BM, D), lambda i: (i, 0)),
            pl.BlockSpec((BM, 1), lambda i: (i, 0)),
        ],
        out_shape=[
            jax.ShapeDtypeStruct((NP, D), jnp.float32),
            jax.ShapeDtypeStruct((NP, 1), jnp.float32),
        ],
    )(xp, W, d0, d1)


def _finish_call(p0, p1, hs, dinv, b2, w2, BM):
    NP, D = hs.shape

    def body(p0_ref, p1_ref, hs_ref, dinv_ref, b_ref, w_ref, o_ref):
        s = p0_ref[...] + p1_ref[...] + hs_ref[...]
        out = dinv_ref[...] * s + b_ref[...]
        o_ref[...] = jnp.where(out > 0, out, w_ref[...] * out)

    return pl.pallas_call(
        body,
        grid=(NP // BM,),
        in_specs=[
            pl.BlockSpec((BM, D), lambda i: (i, 0)),
            pl.BlockSpec((BM, D), lambda i: (i, 0)),
            pl.BlockSpec((BM, D), lambda i: (i, 0)),
            pl.BlockSpec((BM, 1), lambda i: (i, 0)),
            pl.BlockSpec((1, D), lambda i: (0, 0)),
            pl.BlockSpec((1, D), lambda i: (0, 0)),
        ],
        out_specs=pl.BlockSpec((BM, D), lambda i: (i, 0)),
        out_shape=jax.ShapeDtypeStruct((NP, D), jnp.float32),
    )(p0, p1, hs, dinv, b2, w2)


def kernel(x, edge_index, W, b, prelu_w):
    N, D = x.shape
    E = edge_index.shape[1]
    NP = _cdiv(N, 2048) * 2048
    assert NP > N
    SLAB = NP // NS
    BM = 1024
    EB = _cdiv(E, NW * 128)
    EB = EB + (EB % 2)
    EP = NW * EB * 128

    assert NP <= 128 * 128
    pad = jnp.full((EP - E,), NP - 1, dtype=edge_index.dtype)
    rowp = jnp.concatenate([edge_index[0], pad]).reshape(NW, EB, 128)
    colp = jnp.concatenate([edge_index[1], pad]).reshape(NW, EB, 128)

    zero_h = jnp.zeros((128, 128), jnp.float32)
    idrows = jnp.arange(128, dtype=jnp.int32).reshape(1, 128)
    zrow = jnp.zeros((SLAB, D), jnp.float32)

    degp = _deg_call(colp.reshape(NW, EB * 128), zero_h, idrows)
    d0 = degp[0].reshape(-1)[:NP, None]
    d1 = degp[1].reshape(-1)[:NP, None]

    xp = jnp.pad(x, ((0, NP - N), (0, 0)))
    hs, dinv = _prescale_call(xp, W, d0, d1, BM)

    P = _msg_call(hs, rowp, colp, zrow, NP, D)

    z = _finish_call(P[0], P[1], hs, dinv,
                     b.reshape(1, D), prelu_w.reshape(1, D), BM)
    return z[:N]


# speedup vs baseline: 11.2280x; 1.0665x over previous
import functools

import jax
import jax.numpy as jnp
from jax import lax
from jax.experimental import pallas as pl
from jax.experimental.pallas import tpu as pltpu
from jax.experimental.pallas import tpu_sc as plsc

NC = 2
NS = 16
NW = NC * NS
LANES = 16


def _cdiv(a, b):
    return (a + b - 1) // b


def _deg_call(colf, zero_h, idrows):
    EC = colf.shape[1]
    mesh = plsc.VectorSubcoreMesh(core_axis_name="c", subcore_axis_name="s")

    @functools.partial(
        pl.kernel,
        out_type=jax.ShapeDtypeStruct((NC, 128, 128), jnp.float32),
        mesh=mesh,
        scratch_types=[
            pltpu.VMEM((EC,), jnp.int32),
            pltpu.VMEM((128, 128), jnp.float32),
            pltpu.VMEM((1, 128), jnp.int32),
            pltpu.VMEM_SHARED((128, 128), jnp.float32),
        ],
        compiler_params=pltpu.CompilerParams(needs_layout_passes=False),
    )
    def deg_kernel(colf_hbm, zero_hbm, idr_hbm, deg_hbm,
                   col_v, hist_v, idr_v, acc_sh):
        cid = lax.axis_index("c")
        sid = lax.axis_index("s")
        w = cid * NS + sid
        pltpu.sync_copy(colf_hbm.at[w], col_v)
        pltpu.sync_copy(zero_hbm, hist_v)
        pltpu.sync_copy(idr_hbm, idr_v)
        pltpu.sync_copy(zero_hbm.at[pl.ds(0, 8)], acc_sh.at[pl.ds(sid * 8, 8)])

        ones16 = jnp.full((16,), 1.0, jnp.float32)

        def body(g, carry):
            idx16 = col_v[pl.ds(g * 16, 16)]
            hi = lax.shift_right_logical(idx16, 7)
            lo = lax.bitwise_and(idx16, 127)
            plsc.addupdate_scatter(hist_v, [hi, lo], ones16)
            return carry

        lax.fori_loop(0, EC // 16, body, 0)
        plsc.subcore_barrier()
        pltpu.sync_copy(hist_v, acc_sh.at[idr_v.at[0]], add=True)
        plsc.subcore_barrier()
        pltpu.sync_copy(acc_sh.at[pl.ds(sid * 8, 8)],
                        deg_hbm.at[cid, pl.ds(sid * 8, 8)])

    return deg_kernel(colf, zero_h, idrows)


def _msg_call(hs, rowr, colr, zrow, NP, D):
    EB = rowr.shape[1]
    SLAB = NP // NS
    mesh = plsc.VectorSubcoreMesh(core_axis_name="c", subcore_axis_name="s")

    CH = 8
    assert EB % (2 * CH) == 0
    NCHH = EB // CH // 2

    @functools.partial(
        pl.kernel,
        out_type=jax.ShapeDtypeStruct((NC, NP, D), jnp.float32),
        mesh=mesh,
        scratch_types=[
            pltpu.VMEM((2, CH, 128), jnp.int32),
            pltpu.VMEM((2, CH, 128), jnp.int32),
            pltpu.VMEM((128, D), jnp.float32),
            pltpu.VMEM((128, D), jnp.float32),
            pltpu.VMEM_SHARED((NP, D), jnp.float32),
            pltpu.SemaphoreType.DMA,
            pltpu.SemaphoreType.DMA,
            pltpu.SemaphoreType.DMA,
            pltpu.SemaphoreType.DMA,
        ],
    )
    def msg_kernel(hs_hbm, rowr_hbm, colr_hbm, zrow_hbm, out_hbm,
                   rowc, colc, m0, m1, acc_sh, g0, g1, i0, i1):
        bufs = (m0, m1)
        gs = (g0, g1)
        isems = (i0, i1)
        cid = lax.axis_index("c")
        sid = lax.axis_index("s")
        w = cid * NS + sid

        def refill(c, slot):
            pltpu.async_copy(rowr_hbm.at[w, pl.ds(c * CH, CH)],
                             rowc.at[slot], isems[slot])
            pltpu.async_copy(colr_hbm.at[w, pl.ds(c * CH, CH)],
                             colc.at[slot], isems[slot])

        def wait_refill(slot):
            pltpu.make_async_copy(rowr_hbm.at[w, pl.ds(0, CH)],
                                  rowc.at[slot], isems[slot]).wait()
            pltpu.make_async_copy(colr_hbm.at[w, pl.ds(0, CH)],
                                  colc.at[slot], isems[slot]).wait()

        def gather(slot, k, b):
            pltpu.async_copy(hs_hbm.at[rowc.at[slot, k]], bufs[b], gs[b])

        def wait_gather(b):
            pltpu.make_async_copy(hs_hbm.at[rowc.at[0, 0]],
                                  bufs[b], gs[b]).wait()

        refill(0, 0)
        refill(1, 1)
        pltpu.sync_copy(zrow_hbm, acc_sh.at[pl.ds(sid * SLAB, SLAB)])
        plsc.subcore_barrier()
        wait_refill(0)
        gather(0, 0, 0)
        gather(0, 1, 1)

        def super_body(g, carry):
            for cc in range(2):
                c = g * 2 + cc
                for k in range(CH):
                    wait_gather(k & 1)
                    pltpu.sync_copy(bufs[k & 1], acc_sh.at[colc.at[cc, k]],
                                    add=True)
                    if k == CH - 3:
                        wait_refill(1 - cc)
                    if k < CH - 2:
                        gather(cc, k + 2, k & 1)
                    else:
                        gather(1 - cc, k - (CH - 2), k & 1)
                refill(jnp.minimum(c + 2, 2 * NCHH - 1), cc)
            return carry

        lax.fori_loop(0, NCHH, super_body, 0)
        wait_gather(0)
        wait_gather(1)
        wait_refill(1)
        plsc.subcore_barrier()
        pltpu.sync_copy(acc_sh.at[pl.ds(sid * SLAB, SLAB)],
                        out_hbm.at[cid, pl.ds(sid * SLAB, SLAB)])

    return msg_kernel(hs, rowr, colr, zrow)


def _prescale_call(xp, W, d0, d1, BM):
    NP, D = xp.shape

    def body(x_ref, w_ref, d0_ref, d1_ref, hs_ref, dinv_ref):
        deg = d0_ref[...] + d1_ref[...] + 1.0
        dinv = lax.rsqrt(deg)
        h = jnp.dot(x_ref[...], w_ref[...], preferred_element_type=jnp.float32,
                    precision=lax.Precision.HIGHEST)
        hs_ref[...] = h * dinv
        dinv_ref[...] = dinv

    return pl.pallas_call(
        body,
        grid=(NP // BM,),
        in_specs=[
            pl.BlockSpec((BM, D), lambda i: (i, 0)),
            pl.BlockSpec((D, D), lambda i: (0, 0)),
            pl.BlockSpec((BM, 1), lambda i: (i, 0)),
            pl.BlockSpec((BM, 1), lambda i: (i, 0)),
        ],
        out_specs=[
            pl.BlockSpec((BM, D), lambda i: (i, 0)),
            pl.BlockSpec((BM, 1), lambda i: (i, 0)),
        ],
        out_shape=[
            jax.ShapeDtypeStruct((NP, D), jnp.float32),
            jax.ShapeDtypeStruct((NP, 1), jnp.float32),
        ],
    )(xp, W, d0, d1)


def _finish_call(p0, p1, hs, dinv, b2, w2, BM):
    NP, D = hs.shape

    def body(p0_ref, p1_ref, hs_ref, dinv_ref, b_ref, w_ref, o_ref):
        s = p0_ref[...] + p1_ref[...] + hs_ref[...]
        out = dinv_ref[...] * s + b_ref[...]
        o_ref[...] = jnp.where(out > 0, out, w_ref[...] * out)

    return pl.pallas_call(
        body,
        grid=(NP // BM,),
        in_specs=[
            pl.BlockSpec((BM, D), lambda i: (i, 0)),
            pl.BlockSpec((BM, D), lambda i: (i, 0)),
            pl.BlockSpec((BM, D), lambda i: (i, 0)),
            pl.BlockSpec((BM, 1), lambda i: (i, 0)),
            pl.BlockSpec((1, D), lambda i: (0, 0)),
            pl.BlockSpec((1, D), lambda i: (0, 0)),
        ],
        out_specs=pl.BlockSpec((BM, D), lambda i: (i, 0)),
        out_shape=jax.ShapeDtypeStruct((NP, D), jnp.float32),
    )(p0, p1, hs, dinv, b2, w2)


def kernel(x, edge_index, W, b, prelu_w):
    N, D = x.shape
    E = edge_index.shape[1]
    NP = _cdiv(N, 2048) * 2048
    assert NP > N
    SLAB = NP // NS
    BM = 1024
    EB = _cdiv(E, NW * 128)
    EB = EB + (EB % 2)
    EP = NW * EB * 128

    assert NP <= 128 * 128
    pad = jnp.full((EP - E,), NP - 1, dtype=edge_index.dtype)
    rowp = jnp.concatenate([edge_index[0], pad]).reshape(NW, EB, 128)
    colp = jnp.concatenate([edge_index[1], pad]).reshape(NW, EB, 128)

    zero_h = jnp.zeros((128, 128), jnp.float32)
    idrows = jnp.arange(128, dtype=jnp.int32).reshape(1, 128)
    zrow = jnp.zeros((SLAB, D), jnp.float32)

    degp = _deg_call(colp.reshape(NW, EB * 128), zero_h, idrows)
    d0 = degp[0].reshape(-1)[:NP, None]
    d1 = degp[1].reshape(-1)[:NP, None]

    xp = jnp.pad(x, ((0, NP - N), (0, 0)))
    hs, dinv = _prescale_call(xp, W, d0, d1, BM)

    P = _msg_call(hs, rowp, colp, zrow, NP, D)

    z = _finish_call(P[0], P[1], hs, dinv,
                     b.reshape(1, D), prelu_w.reshape(1, D), BM)
    return z[:N]
